# baseline (speedup 1.0000x reference)
import functools

import jax
import jax.numpy as jnp
from jax import lax
from jax.experimental import pallas as pl
from jax.experimental.pallas import tpu as pltpu
from jax.experimental.pallas import tpu_sc as plsc

VOCAB = 1000
EMBED_DIM = 64
BATCH = 4096
HIST = 50

_NC = 2
_NS = 16
_NW = _NC * _NS

_B_PER_W = BATCH // _NW
_NBUF = 8
_GROUPS = _B_PER_W // _NBUF
_ROW = HIST * EMBED_DIM
_FLAT = BATCH * _ROW


@functools.partial(
    pl.kernel,
    mesh=plsc.VectorSubcoreMesh(core_axis_name="c", subcore_axis_name="s"),
    out_type=jax.ShapeDtypeStruct((BATCH, HIST, EMBED_DIM), jnp.float32),
    scratch_types=[
        pltpu.VMEM((_B_PER_W, HIST), jnp.int32),
        pltpu.VMEM((_NBUF, HIST, EMBED_DIM), jnp.float32),
    ] + [pltpu.SemaphoreType.DMA] * (2 * _NBUF),
    compiler_params=pltpu.CompilerParams(use_tc_tiling_on_sc=False),
)
def _gather_kernel(x_hbm, table_hbm, out_hbm, idx_v, rows_v, *sems):
    gsems = sems[:_NBUF]
    wsems = sems[_NBUF:]
    wid = lax.axis_index("s") * _NC + lax.axis_index("c")
    base = wid * _B_PER_W

    pltpu.sync_copy(x_hbm.at[pl.ds(base, _B_PER_W)], idx_v)

    def body(g, carry):
        for u in range(_NBUF):
            c = g * _NBUF + u

            @pl.when(g > 0)
            def _():
                pltpu.make_async_copy(
                    rows_v.at[u],
                    out_hbm.at[base],
                    wsems[u],
                ).wait()

            pltpu.async_copy(
                table_hbm.at[idx_v.at[c]],
                rows_v.at[u],
                gsems[u],
            )
        for u in range(_NBUF):
            c = g * _NBUF + u
            pltpu.make_async_copy(
                table_hbm.at[idx_v.at[c]],
                rows_v.at[u],
                gsems[u],
            ).wait()
            pltpu.async_copy(
                rows_v.at[u],
                out_hbm.at[base + c],
                wsems[u],
            )
        return carry

    lax.fori_loop(0, _GROUPS, body, 0)

    for u in range(_NBUF):
        pltpu.make_async_copy(
            rows_v.at[u],
            out_hbm.at[base],
            wsems[u],
        ).wait()


_KB = 16
_IN_ROWS = _KB * _ROW // 128


def _relayout_body(in_ref, out_ref):
    for b in range(_KB):
        rawb = in_ref[pl.ds(b * 25, 25), :]
        out_ref[b, 0:HIST:2, :] = rawb[:, 0:EMBED_DIM]
        out_ref[b, 1:HIST:2, :] = rawb[:, EMBED_DIM : 2 * EMBED_DIM]


def _relayout(flat2d):
    return pl.pallas_call(
        _relayout_body,
        grid=(BATCH // _KB,),
        in_specs=[pl.BlockSpec((_IN_ROWS, 128), lambda g: (g, 0))],
        out_specs=pl.BlockSpec((_KB, HIST, EMBED_DIM), lambda g: (g, 0, 0)),
        out_shape=jax.ShapeDtypeStruct((BATCH, HIST, EMBED_DIM), jnp.float32),
        compiler_params=pltpu.CompilerParams(
            dimension_semantics=("arbitrary",),
        ),
    )(flat2d)


def kernel(x, table):
    flat = _gather_kernel(x, table)
    return _relayout(flat.reshape(_FLAT // 128, 128))


# speedup vs baseline: 2.2217x; 2.2217x over previous
import functools

import jax
import jax.numpy as jnp
from jax import lax
from jax.experimental import pallas as pl
from jax.experimental.pallas import tpu as pltpu
from jax.experimental.pallas import tpu_sc as plsc

VOCAB = 1000
EMBED_DIM = 64
BATCH = 4096
HIST = 50

_NC = 2
_NS = 16
_NW = _NC * _NS

_BC = 128
_NCHUNK = BATCH // _BC
_DG = 8
_HN = 13


@functools.partial(
    pl.kernel,
    mesh=plsc.VectorSubcoreMesh(core_axis_name="c", subcore_axis_name="s"),
    out_type=jax.ShapeDtypeStruct((HIST, _DG, BATCH // _BC, 8, _BC), jnp.float32),
    scratch_types=[
        pltpu.VMEM((VOCAB * EMBED_DIM,), jnp.float32),
        pltpu.VMEM((2, HIST, _BC), jnp.int32),
        pltpu.VMEM((2, _HN, _DG, _BC), jnp.float32),
        pltpu.SemaphoreType.DMA,
        pltpu.SemaphoreType.DMA,
        pltpu.SemaphoreType.DMA,
        pltpu.SemaphoreType.DMA,
    ],
    compiler_params=pltpu.CompilerParams(
        use_tc_tiling_on_sc=False, needs_layout_passes=False
    ),
)
def _tgather_kernel(xtr_hbm, tflat_hbm, out_hbm, tab_v, idx_v, buf_v,
                    isem0, isem1, wsem0, wsem1):
    isems = (isem0, isem1)
    wsems = (wsem0, wsem1)
    wid = lax.axis_index("s") * _NC + lax.axis_index("c")
    dg = lax.rem(wid, _DG)
    hq = wid // _DG
    d0 = pl.multiple_of(dg * 8, 8)
    h0 = hq * _HN - 2 * (hq // 3)

    pltpu.sync_copy(tflat_hbm, tab_v)
    pltpu.async_copy(xtr_hbm.at[0], idx_v.at[0], isems[0])
    pltpu.async_copy(xtr_hbm.at[1], idx_v.at[1], isems[1])

    def compute_chunk(j, p):
        def hh_body(hh, carry):
            for bl in range(_BC // 16):
                iv = idx_v[p, h0 + hh, pl.ds(bl * 16, 16)]
                for dd in range(_DG):
                    addr = iv + (d0 + dd) * VOCAB
                    buf_v[p, hh, dd, pl.ds(bl * 16, 16)] = plsc.load_gather(
                        tab_v, [addr]
                    )
            return carry

        lax.fori_loop(0, _HN, hh_body, 0)

    def body(g, carry):
        for p in range(2):
            j = 2 * g + p
            b0 = pl.multiple_of(j * _BC, _BC)
            pltpu.make_async_copy(xtr_hbm.at[0], idx_v.at[p], isems[p]).wait()
            @pl.when(g > 0)
            def _():
                pltpu.make_async_copy(
                    buf_v.at[p],
                    out_hbm.at[pl.ds(0, _HN), 0, 0],
                    wsems[p],
                ).wait()

            compute_chunk(j, p)

            pltpu.async_copy(
                buf_v.at[p],
                out_hbm.at[pl.ds(h0, _HN), dg, j],
                wsems[p],
            )

            @pl.when(g < _NCHUNK // 2 - 1)
            def _():
                pltpu.async_copy(xtr_hbm.at[j + 2], idx_v.at[p], isems[p])

        return carry

    lax.fori_loop(0, _NCHUNK // 2, body, 0)

    for p in range(2):
        pltpu.make_async_copy(
            buf_v.at[p],
            out_hbm.at[pl.ds(0, _HN), 0, 0],
            wsems[p],
        ).wait()


def kernel(x, table):
    tflat = table.T.reshape(-1)
    xtr = x.T.reshape(HIST, _NCHUNK, _BC).transpose(1, 0, 2)
    t5 = _tgather_kernel(xtr, tflat)
    return t5.transpose(2, 4, 0, 1, 3).reshape(BATCH, HIST, EMBED_DIM)
